# baseline (speedup 1.0000x reference)
import functools

import jax
import jax.numpy as jnp
from jax import lax
from jax.experimental import pallas as pl
from jax.experimental.pallas import tpu as pltpu
from jax.experimental.pallas import tpu_sc as plsc

BATCH = 16384
EMBED = 64
NUSER = 1000000
NMOVIE = 100000

_NC, _NS, _L = 2, 16, 16
_NW = _NC * _NS
_BPW = BATCH // _NW
_CHUNK = 128
_NCHUNK = _BPW // _CHUNK
_BLOCKS = _CHUNK // 16

_TC_BN = 16384
_SC_BLKS = 16
_SPLIT = _SC_BLKS * _TC_BN
_SPLIT_R = _SPLIT // 16
_TC_ROWS = NUSER - _SPLIT
_TC_R = _TC_ROWS // 16
_CPW = _SPLIT // _NW
_SUP = 512
_NSUP = _CPW // _SUP


def _tc_dot_body(w_ref, x_ref, o_ref):
    o_ref[...] = jnp.dot(w_ref[...], x_ref[...],
                         preferred_element_type=jnp.float32)[0]


def _tc_dot(w_half, table_t, n, off):
    grid = ((n - off * _TC_BN) + _TC_BN - 1) // _TC_BN
    return pl.pallas_call(
        _tc_dot_body,
        grid=(grid,),
        in_specs=[
            pl.BlockSpec((1, EMBED), lambda i: (0, 0)),
            pl.BlockSpec((EMBED, _TC_BN), lambda i: (0, i + off)),
        ],
        out_specs=pl.BlockSpec((_TC_BN,), lambda i: (i,)),
        out_shape=jax.ShapeDtypeStruct((n - off * _TC_BN,), jnp.float32),
    )(w_half, table_t)


def _sc_dot_body(ut_hbm, wb_hbm, out_hbm, jbuf, wv, outv, sem):
    wid = lax.axis_index("s") * _NC + lax.axis_index("c")
    base = wid * _CPW
    pltpu.sync_copy(wb_hbm, wv)

    def issue(sup, p):
        col = base + sup * _SUP
        for J in range(8):
            pltpu.async_copy(ut_hbm.at[pl.ds(8 * J, 8), pl.ds(col, _SUP)],
                             jbuf.at[p, J], sem.at[p])

    def drain(p):
        for J in range(8):
            pltpu.make_async_copy(ut_hbm.at[pl.ds(0, 8), pl.ds(0, _SUP)],
                                  jbuf.at[p, J], sem.at[p]).wait()

    issue(0, 0)
    issue(1, 1)

    def pair_body(i, carry):
        for p in range(2):
            cur = 2 * i + p
            drain(p)

            def blk_body(b, c2, p=p, cur=cur):
                col0 = b * 128
                for m in range(8):
                    acc = None
                    for J in range(8):
                        for r in range(8):
                            v = (jbuf[p, J, r, pl.ds(col0 + 16 * m, 16)]
                                 * wv[8 * J + r, pl.ds(0, 16)])
                            acc = v if acc is None else acc + v
                    outv[pl.ds(cur * _SUP + col0 + 16 * m, 16)] = acc
                return c2

            lax.fori_loop(0, _SUP // 128, blk_body, 0)

            @pl.when(cur < _NSUP - 2)
            def _(p=p, cur=cur):
                issue(cur + 2, p)
        return carry

    lax.fori_loop(0, _NSUP // 2, pair_body, 0)
    pltpu.sync_copy(outv, out_hbm.at[pl.ds(base, _CPW)])


def _sc_dot(table_t, wb):
    mesh = plsc.VectorSubcoreMesh(core_axis_name="c", subcore_axis_name="s")
    fn = pl.kernel(
        _sc_dot_body,
        mesh=mesh,
        compiler_params=pltpu.CompilerParams(use_tc_tiling_on_sc=True,
                                             needs_layout_passes=False),
        out_type=jax.ShapeDtypeStruct((_SPLIT,), jnp.float32),
        scratch_types=[
            pltpu.VMEM((2, 8, 8, _SUP), jnp.float32),
            pltpu.VMEM((EMBED, 16), jnp.float32),
            pltpu.VMEM((_CPW,), jnp.float32),
            pltpu.SemaphoreType.DMA((2,)),
        ],
    )
    return fn(table_t, wb)


def _sc_gather_body(user_hbm, movie_hbm, pa_hbm, pb_hbm, pm_hbm, b_hbm,
                    out_hbm, uidx, midx, urida, uridb, mrid,
                    urowsa, urowsb, mrows, bv, outv, sem):
    wid = lax.axis_index("s") * _NC + lax.axis_index("c")
    base = wid * _BPW

    pltpu.sync_copy(b_hbm, bv)
    for j in range(_NCHUNK):
        pltpu.sync_copy(user_hbm.at[pl.ds(base + j * _CHUNK, _CHUNK)],
                        uidx.at[j])
        pltpu.sync_copy(movie_hbm.at[pl.ds(base + j * _CHUNK, _CHUNK)],
                        midx.at[j])

    zero = jnp.zeros((16,), dtype=jnp.int32)
    ca = jnp.full((16,), _SPLIT_R - 1, dtype=jnp.int32)
    cb = jnp.full((16,), _SPLIT_R, dtype=jnp.int32)
    for j in range(_NCHUNK):
        for b in range(_BLOCKS):
            s = pl.ds(16 * b, 16)
            ur = lax.shift_right_logical(uidx[j, s], 4)
            urida[j, s] = jnp.minimum(ur, ca)
            uridb[j, s] = jnp.maximum(ur - cb, zero)
            mrid[j, s] = lax.shift_right_logical(midx[j, s], 4)

    copies = []
    for j in range(_NCHUNK):
        copies.append(pltpu.async_copy(pa_hbm.at[urida.at[j]], urowsa.at[j],
                                       sem.at[j]))
        copies.append(pltpu.async_copy(pb_hbm.at[uridb.at[j]], urowsb.at[j],
                                       sem.at[j]))
        copies.append(pltpu.async_copy(pm_hbm.at[mrid.at[j]], mrows.at[j],
                                       sem.at[j]))

    bias = bv[...]
    ii = lax.iota(jnp.int32, 16)
    mask15 = jnp.full((16,), 15, dtype=jnp.int32)
    split = jnp.full((16,), _SPLIT, dtype=jnp.int32)

    for j in range(_NCHUNK):
        copies[3 * j].wait()
        copies[3 * j + 1].wait()
        copies[3 * j + 2].wait()
        for b in range(_BLOCKS):
            s = pl.ds(16 * b, 16)
            row = ii + (16 * b)
            u = uidx[j, s]
            ga = plsc.load_gather(urowsa.at[j], [row, u & mask15])
            gb = plsc.load_gather(urowsb.at[j], [row, u & mask15])
            gm = plsc.load_gather(mrows.at[j], [row, midx[j, s] & mask15])
            gu = jnp.where(u < split, ga, gb)
            outv[pl.ds(j * _CHUNK + 16 * b, 16)] = gu + gm + bias

    pltpu.sync_copy(outv, out_hbm.at[pl.ds(base, _BPW)])


def _sc_gather(user, movie, pa2, pb2, pm2, b_vec):
    mesh = plsc.VectorSubcoreMesh(core_axis_name="c", subcore_axis_name="s")
    fn = pl.kernel(
        _sc_gather_body,
        mesh=mesh,
        compiler_params=pltpu.CompilerParams(use_tc_tiling_on_sc=False,
                                             needs_layout_passes=False),
        out_type=jax.ShapeDtypeStruct((BATCH,), jnp.float32),
        scratch_types=[
            pltpu.VMEM((_NCHUNK, _CHUNK), jnp.int32),
            pltpu.VMEM((_NCHUNK, _CHUNK), jnp.int32),
            pltpu.VMEM((_NCHUNK, _CHUNK), jnp.int32),
            pltpu.VMEM((_NCHUNK, _CHUNK), jnp.int32),
            pltpu.VMEM((_NCHUNK, _CHUNK), jnp.int32),
            pltpu.VMEM((_NCHUNK, _CHUNK, 16), jnp.float32),
            pltpu.VMEM((_NCHUNK, _CHUNK, 16), jnp.float32),
            pltpu.VMEM((_NCHUNK, _CHUNK, 16), jnp.float32),
            pltpu.VMEM((16,), jnp.float32),
            pltpu.VMEM((_BPW,), jnp.float32),
            pltpu.SemaphoreType.DMA((_NCHUNK,)),
        ],
    )
    return fn(user, movie, pa2, pb2, pm2, b_vec)


@jax.jit
def _call(user, movie, user_table, movie_table, w_vec, b_vec, wb_u):
    ut_t = user_table.T
    pu_sc = _sc_dot(ut_t, wb_u)
    pu_tc = _tc_dot(w_vec[:EMBED].reshape(1, EMBED), ut_t, NUSER, _SC_BLKS)
    pm = _tc_dot(w_vec[EMBED:].reshape(1, EMBED), movie_table.T, NMOVIE, 0)
    pa2 = pu_sc.reshape(_SPLIT_R, 16)
    pb2 = pu_tc.reshape(_TC_R, 16)
    pm2 = pm.reshape(NMOVIE // 16, 16)
    return _sc_gather(user, movie, pa2, pb2, pm2, b_vec)


def kernel(user, movie, user_table, movie_table, fc_w, fc_b):
    w_vec = fc_w.reshape(2 * EMBED).astype(jnp.float32)
    b_vec = jnp.broadcast_to(fc_b.astype(jnp.float32), (16,))
    wb_u = jnp.tile(w_vec[:EMBED, None], (1, 16))
    out = _call(user.astype(jnp.int32), movie.astype(jnp.int32),
                user_table, movie_table, w_vec, b_vec, wb_u)
    return out.reshape(BATCH, 1)


# speedup vs baseline: 1.4289x; 1.4289x over previous
import functools

import jax
import jax.numpy as jnp
from jax import lax
from jax.experimental import pallas as pl
from jax.experimental.pallas import tpu as pltpu
from jax.experimental.pallas import tpu_sc as plsc

BATCH = 16384
EMBED = 64
NUSER = 1000000
NMOVIE = 100000

_NC, _NS, _L = 2, 16, 16
_NW = _NC * _NS
_BPW = BATCH // _NW
_CHUNK = 128
_NCHUNK = _BPW // _CHUNK
_BLOCKS = _CHUNK // 16

_TC_BN = 16384
_SC_BLKS = 16
_SPLIT = _SC_BLKS * _TC_BN
_SPLIT_R = _SPLIT // 16
_TC_ROWS = NUSER - _SPLIT
_TC_R = _TC_ROWS // 16
_CPW = _SPLIT // _NW
_SUP = 512
_NSUP = _CPW // _SUP


def _tc_dot_body(w_ref, x_ref, o_ref):
    o_ref[...] = jnp.dot(w_ref[...], x_ref[...],
                         preferred_element_type=jnp.float32)[0]


def _tc_dot(w_half, table_t, n, off):
    grid = ((n - off * _TC_BN) + _TC_BN - 1) // _TC_BN
    return pl.pallas_call(
        _tc_dot_body,
        grid=(grid,),
        in_specs=[
            pl.BlockSpec((1, EMBED), lambda i: (0, 0)),
            pl.BlockSpec((EMBED, _TC_BN), lambda i: (0, i + off)),
        ],
        out_specs=pl.BlockSpec((_TC_BN,), lambda i: (i,)),
        out_shape=jax.ShapeDtypeStruct((n - off * _TC_BN,), jnp.float32),
    )(w_half, table_t)


def _sc_dot_body(ut_hbm, wb_hbm, out_hbm, jbuf, wv, outv, sem):
    wid = lax.axis_index("s") * _NC + lax.axis_index("c")
    base = wid * _CPW
    pltpu.sync_copy(wb_hbm, wv)

    def issue(sup, p):
        col = base + sup * _SUP
        for J in range(8):
            pltpu.async_copy(ut_hbm.at[pl.ds(8 * J, 8), pl.ds(col, _SUP)],
                             jbuf.at[p, J], sem.at[p])

    def drain(p):
        for J in range(8):
            pltpu.make_async_copy(ut_hbm.at[pl.ds(0, 8), pl.ds(0, _SUP)],
                                  jbuf.at[p, J], sem.at[p]).wait()

    issue(0, 0)
    issue(1, 1)

    def pair_body(i, carry):
        for p in range(2):
            cur = 2 * i + p
            drain(p)

            def blk_body(b, c2, p=p, cur=cur):
                col0 = b * 128
                for m in range(8):
                    acc = None
                    for J in range(8):
                        for r in range(8):
                            v = (jbuf[p, J, r, pl.ds(col0 + 16 * m, 16)]
                                 * wv[8 * J + r, pl.ds(0, 16)])
                            acc = v if acc is None else acc + v
                    outv[pl.ds(cur * _SUP + col0 + 16 * m, 16)] = acc
                return c2

            lax.fori_loop(0, _SUP // 128, blk_body, 0)

            @pl.when(cur < _NSUP - 2)
            def _(p=p, cur=cur):
                issue(cur + 2, p)
        return carry

    lax.fori_loop(0, _NSUP // 2, pair_body, 0)
    pltpu.sync_copy(outv, out_hbm.at[pl.ds(base, _CPW)])


def _sc_dot(table_t, wb):
    mesh = plsc.VectorSubcoreMesh(core_axis_name="c", subcore_axis_name="s")
    fn = pl.kernel(
        _sc_dot_body,
        mesh=mesh,
        compiler_params=pltpu.CompilerParams(use_tc_tiling_on_sc=True,
                                             needs_layout_passes=False),
        out_type=jax.ShapeDtypeStruct((_SPLIT,), jnp.float32),
        scratch_types=[
            pltpu.VMEM((2, 8, 8, _SUP), jnp.float32),
            pltpu.VMEM((EMBED, 16), jnp.float32),
            pltpu.VMEM((_CPW,), jnp.float32),
            pltpu.SemaphoreType.DMA((2,)),
        ],
    )
    return fn(table_t, wb)


def _sc_gather_body(user_hbm, movie_hbm, pu_hbm, pm_hbm, b_hbm,
                    out_hbm, uidx, midx, urid, mrid,
                    urows, mrows, bv, outv, sem):
    wid = lax.axis_index("s") * _NC + lax.axis_index("c")
    base = wid * _BPW

    pltpu.sync_copy(b_hbm, bv)
    for j in range(_NCHUNK):
        pltpu.sync_copy(user_hbm.at[pl.ds(base + j * _CHUNK, _CHUNK)],
                        uidx.at[j])
        pltpu.sync_copy(movie_hbm.at[pl.ds(base + j * _CHUNK, _CHUNK)],
                        midx.at[j])

    for j in range(_NCHUNK):
        for b in range(_BLOCKS):
            s = pl.ds(16 * b, 16)
            urid[j, s] = lax.shift_right_logical(uidx[j, s], 4)
            mrid[j, s] = lax.shift_right_logical(midx[j, s], 4)

    copies = []
    for j in range(_NCHUNK):
        copies.append(pltpu.async_copy(pu_hbm.at[urid.at[j]], urows.at[j],
                                       sem.at[j]))
        copies.append(pltpu.async_copy(pm_hbm.at[mrid.at[j]], mrows.at[j],
                                       sem.at[j]))

    bias = bv[...]
    ii = lax.iota(jnp.int32, 16)
    mask15 = jnp.full((16,), 15, dtype=jnp.int32)

    for j in range(_NCHUNK):
        copies[2 * j].wait()
        copies[2 * j + 1].wait()
        for b in range(_BLOCKS):
            s = pl.ds(16 * b, 16)
            row = ii + (16 * b)
            gu = plsc.load_gather(urows.at[j], [row, uidx[j, s] & mask15])
            gm = plsc.load_gather(mrows.at[j], [row, midx[j, s] & mask15])
            outv[pl.ds(j * _CHUNK + 16 * b, 16)] = gu + gm + bias

    pltpu.sync_copy(outv, out_hbm.at[pl.ds(base, _BPW)])


def _sc_gather(user, movie, pu2, pm2, b_vec):
    mesh = plsc.VectorSubcoreMesh(core_axis_name="c", subcore_axis_name="s")
    fn = pl.kernel(
        _sc_gather_body,
        mesh=mesh,
        compiler_params=pltpu.CompilerParams(use_tc_tiling_on_sc=False,
                                             needs_layout_passes=False),
        out_type=jax.ShapeDtypeStruct((BATCH,), jnp.float32),
        scratch_types=[
            pltpu.VMEM((_NCHUNK, _CHUNK), jnp.int32),
            pltpu.VMEM((_NCHUNK, _CHUNK), jnp.int32),
            pltpu.VMEM((_NCHUNK, _CHUNK), jnp.int32),
            pltpu.VMEM((_NCHUNK, _CHUNK), jnp.int32),
            pltpu.VMEM((_NCHUNK, _CHUNK, 16), jnp.float32),
            pltpu.VMEM((_NCHUNK, _CHUNK, 16), jnp.float32),
            pltpu.VMEM((16,), jnp.float32),
            pltpu.VMEM((_BPW,), jnp.float32),
            pltpu.SemaphoreType.DMA((_NCHUNK,)),
        ],
    )
    return fn(user, movie, pu2, pm2, b_vec)


@jax.jit
def _call(user, movie, user_table, movie_table, w_vec, b_vec, wb_u):
    ut_t = user_table.T
    pu_sc = _sc_dot(ut_t, wb_u)
    pu_tc = _tc_dot(w_vec[:EMBED].reshape(1, EMBED), ut_t, NUSER, _SC_BLKS)
    pm = _tc_dot(w_vec[EMBED:].reshape(1, EMBED), movie_table.T, NMOVIE, 0)
    pu2 = jnp.concatenate([pu_sc, pu_tc]).reshape(NUSER // 16, 16)
    pm2 = pm.reshape(NMOVIE // 16, 16)
    return _sc_gather(user, movie, pu2, pm2, b_vec)


def kernel(user, movie, user_table, movie_table, fc_w, fc_b):
    w_vec = fc_w.reshape(2 * EMBED).astype(jnp.float32)
    b_vec = jnp.broadcast_to(fc_b.astype(jnp.float32), (16,))
    wb_u = jnp.tile(w_vec[:EMBED, None], (1, 16))
    out = _call(user.astype(jnp.int32), movie.astype(jnp.int32),
                user_table, movie_table, w_vec, b_vec, wb_u)
    return out.reshape(BATCH, 1)
